# baseline (speedup 1.0000x reference)
import jax
import jax.numpy as jnp
from jax.experimental import pallas as pl

N = 10000
G = 16


def _mlp_body(pooled_ref, Wa_ref, ba_ref, Wb_ref, bb_ref, out_ref):
    h = jnp.maximum(pooled_ref[...] @ Wa_ref[...] + ba_ref[...], 0.0)
    out_ref[...] = h @ Wb_ref[...] + bb_ref[...]


def kernel(x, edge_index, edge_weight, batch, W_lin1, b_lin1, W1_rel, b1_rel, W1_root, W2_rel, b2_rel, W2_root, W3_rel, b3_rel, W3_root, W_l2a, b_l2a, W_l2b, b_l2b):
    src = edge_index[0]
    dst = edge_index[1]

    def conv(h, W_rel, b_rel, W_root):
        msg = jnp.take(h, src, axis=0) * edge_weight[:, None]
        agg = jax.ops.segment_sum(msg, dst, num_segments=N)
        return agg @ W_rel + b_rel + h @ W_root

    h = jax.nn.relu(x @ W_lin1 + b_lin1)
    h = jax.nn.relu(conv(h, W1_rel, b1_rel, W1_root))
    h = jax.nn.relu(conv(h, W2_rel, b2_rel, W2_root))
    h = conv(h, W3_rel, b3_rel, W3_root)
    sums = jax.ops.segment_sum(h, batch, num_segments=G)
    cnt = jax.ops.segment_sum(jnp.ones((N,), dtype=jnp.float32), batch, num_segments=G)
    pooled = sums / jnp.maximum(cnt, 1.0)[:, None]

    out = pl.pallas_call(
        _mlp_body,
        out_shape=jax.ShapeDtypeStruct((G, W_l2b.shape[1]), jnp.float32),
    )(pooled, W_l2a, b_l2a, W_l2b, b_l2b)
    return out


# speedup vs baseline: 4.9128x; 4.9128x over previous
import functools

import jax
import jax.numpy as jnp
from jax import lax
from jax.experimental import pallas as pl
from jax.experimental.pallas import tpu as pltpu
from jax.experimental.pallas import tpu_sc as plsc

N = 10000
E = 320000
G = 16
NC = 2
NS = 16
NW = NC * NS
EPT = E // NW
C = 80
NCHUNK = EPT // C
NP = 10240
RPT = NP // NS


def _make_segsum(F: int):
    mesh = plsc.VectorSubcoreMesh(
        core_axis_name="c", subcore_axis_name="s", num_cores=NC, num_subcores=NS
    )

    @functools.partial(
        pl.kernel,
        out_type=jax.ShapeDtypeStruct((NC * NP, F), jnp.float32),
        mesh=mesh,
        scratch_types=[
            pltpu.VMEM_SHARED((NP, F), jnp.float32),
            pltpu.VMEM((C,), jnp.int32),
            pltpu.VMEM((C,), jnp.int32),
            pltpu.VMEM((C,), jnp.float32),
            pltpu.VMEM((C, F), jnp.float32),
            pltpu.SemaphoreType.DMA,
        ],
        compiler_params=pltpu.CompilerParams(use_tc_tiling_on_sc=False),
    )
    def segsum(p_hbm, src_hbm, dst_hbm, ew_hbm, zero_hbm, out_hbm,
               acc, src_v, dst_v, ew_v, rows_v, sem):
        cid = lax.axis_index("c")
        sid = lax.axis_index("s")
        wid = cid * NS + sid

        pltpu.sync_copy(zero_hbm.at[pl.ds(sid * RPT, RPT)],
                        acc.at[pl.ds(sid * RPT, RPT)])
        plsc.subcore_barrier()

        base = wid * EPT

        def chunk_body(k, carry):
            off = base + k * C
            pltpu.sync_copy(src_hbm.at[pl.ds(off, C)], src_v)
            pltpu.sync_copy(dst_hbm.at[pl.ds(off, C)], dst_v)
            pltpu.sync_copy(ew_hbm.at[pl.ds(off, C)], ew_v)
            pltpu.async_copy(p_hbm.at[src_v], rows_v, sem).wait()
            for g in range(C // 16):
                e0 = g * 16
                ew16 = ew_v[pl.ds(e0, 16)]
                for j in range(16):
                    wv = jnp.take_along_axis(
                        ew16, jnp.full((16,), j, jnp.int32), axis=0)
                    for f0 in range(0, F, 16):
                        rows_v[e0 + j, pl.ds(f0, 16)] = (
                            rows_v[e0 + j, pl.ds(f0, 16)] * wv)
            pltpu.sync_copy(rows_v, acc.at[dst_v], add=True)
            return carry

        lax.fori_loop(0, NCHUNK, chunk_body, 0)
        plsc.subcore_barrier()

        pltpu.sync_copy(acc.at[pl.ds(sid * RPT, RPT)],
                        out_hbm.at[pl.ds(cid * NP + sid * RPT, RPT)])

    return segsum


_segsum16 = _make_segsum(16)
_segsum64 = _make_segsum(64)


def _tc1_body(x_ref, w_ref, b_ref, out_ref):
    out_ref[...] = jnp.maximum(x_ref[...] @ w_ref[...] + b_ref[...], 0.0)


def _tc2_body(parts_ref, h0_ref, w1r_ref, b1r_ref, w1s_ref,
              w2r_ref, b2r_ref, w2s_ref, p2_ref, r2_ref):
    agg = parts_ref[:N, :] + parts_ref[NP:NP + N, :]
    h1 = jnp.maximum(agg @ w1r_ref[...] + b1r_ref[...]
                     + h0_ref[...] @ w1s_ref[...], 0.0)
    p2_ref[...] = h1 @ w2r_ref[...]
    r2_ref[...] = h1 @ w2s_ref[...] + b2r_ref[...]


def _tc3_body(parts_ref, r_ref, w3r_ref, b3r_ref, w3s_ref, p3_ref, r3_ref):
    h2 = jnp.maximum(parts_ref[:N, :] + parts_ref[NP:NP + N, :] + r_ref[...], 0.0)
    p3_ref[...] = h2 @ w3r_ref[...]
    r3_ref[...] = h2 @ w3s_ref[...] + b3r_ref[...]


def _tc4_body(parts_ref, r_ref, batch_ref, wa_ref, ba_ref, wb_ref, bb_ref,
              out_ref):
    h3 = parts_ref[:N, :] + parts_ref[NP:NP + N, :] + r_ref[...]
    gids = lax.broadcasted_iota(jnp.int32, (N, G), 1)
    oh = (batch_ref[...] == gids).astype(jnp.float32)
    cnt = jnp.sum(oh, axis=0, keepdims=True)
    ohs = oh / jnp.maximum(cnt, 1.0)
    pooled = lax.dot_general(ohs, h3, (((0,), (0,)), ((), ())))
    h4 = jnp.maximum(pooled @ wa_ref[...] + ba_ref[...], 0.0)
    out_ref[...] = h4 @ wb_ref[...] + bb_ref[...]


def _tc_call(body, out_shapes, *args):
    return pl.pallas_call(
        body,
        out_shape=out_shapes,
    )(*args)


def kernel(x, edge_index, edge_weight, batch, W_lin1, b_lin1,
           W1_rel, b1_rel, W1_root,
           W2_rel, b2_rel, W2_root,
           W3_rel, b3_rel, W3_root,
           W_l2a, b_l2a, W_l2b, b_l2b):
    src = edge_index[0].astype(jnp.int32)
    dst = edge_index[1].astype(jnp.int32)
    ew = edge_weight.astype(jnp.float32)
    batch2d = batch.astype(jnp.int32).reshape(N, 1)

    W1p = jnp.pad(W_lin1, ((0, 0), (0, 8)))
    b1p = jnp.pad(b_lin1, (0, 8)).reshape(1, 16)
    W1_rel_p = jnp.pad(W1_rel, ((0, 8), (0, 0)))
    W1_root_p = jnp.pad(W1_root, ((0, 8), (0, 0)))

    zero16 = jnp.zeros((NP, 16), jnp.float32)
    zero64 = jnp.zeros((NP, 64), jnp.float32)

    h0p = _tc_call(_tc1_body, jax.ShapeDtypeStruct((N, 16), jnp.float32),
                   x, W1p, b1p)

    parts1 = _segsum16(h0p, src, dst, ew, zero16)

    p2, r2 = _tc_call(
        _tc2_body,
        (jax.ShapeDtypeStruct((N, 64), jnp.float32),
         jax.ShapeDtypeStruct((N, 64), jnp.float32)),
        parts1, h0p, W1_rel_p, b1_rel.reshape(1, 64), W1_root_p,
        W2_rel, b2_rel.reshape(1, 64), W2_root)

    parts2 = _segsum64(p2, src, dst, ew, zero64)

    p3, r3 = _tc_call(
        _tc3_body,
        (jax.ShapeDtypeStruct((N, 64), jnp.float32),
         jax.ShapeDtypeStruct((N, 64), jnp.float32)),
        parts2, r2, W3_rel, b3_rel.reshape(1, 64), W3_root)

    parts3 = _segsum64(p3, src, dst, ew, zero64)

    out = _tc_call(
        _tc4_body,
        jax.ShapeDtypeStruct((G, b_l2b.shape[0]), jnp.float32),
        parts3, r3, batch2d, W_l2a, b_l2a.reshape(1, 32),
        W_l2b, b_l2b.reshape(1, b_l2b.shape[0]))
    return out


# speedup vs baseline: 6.0648x; 1.2345x over previous
import functools

import jax
import jax.numpy as jnp
from jax import lax
from jax.experimental import pallas as pl
from jax.experimental.pallas import tpu as pltpu
from jax.experimental.pallas import tpu_sc as plsc

N = 10000
E = 320000
G = 16
NC = 2
NS = 16
NW = NC * NS
C = 128
NROW = 2560
EP = NROW * C
CPT = NROW // NW
NP = 10240
RPT = NP // NS


def _make_segsum(F: int):
    mesh = plsc.VectorSubcoreMesh(
        core_axis_name="c", subcore_axis_name="s", num_cores=NC, num_subcores=NS
    )

    @functools.partial(
        pl.kernel,
        out_type=jax.ShapeDtypeStruct((NC * NP, F), jnp.float32),
        mesh=mesh,
        scratch_types=[
            pltpu.VMEM_SHARED((NP, F), jnp.float32),
            pltpu.VMEM((CPT, C), jnp.int32),
            pltpu.VMEM((CPT, C), jnp.int32),
            pltpu.VMEM((CPT, C), jnp.float32),
            pltpu.VMEM((C, F), jnp.float32),
            pltpu.VMEM((C, F), jnp.float32),
            pltpu.SemaphoreType.DMA,
            pltpu.SemaphoreType.DMA,
            pltpu.SemaphoreType.DMA,
            pltpu.SemaphoreType.DMA,
        ],
        compiler_params=pltpu.CompilerParams(use_tc_tiling_on_sc=False),
    )
    def segsum(p_hbm, src_hbm, dst_hbm, ew_hbm, zero_hbm, out_hbm,
               acc, src_sl, dst_sl, ew_sl, buf_a, buf_b,
               sga, sgb, ssa, ssb):
        cid = lax.axis_index("c")
        sid = lax.axis_index("s")
        wid = cid * NS + sid

        pltpu.sync_copy(src_hbm.at[pl.ds(wid * CPT, CPT)], src_sl)
        pltpu.sync_copy(dst_hbm.at[pl.ds(wid * CPT, CPT)], dst_sl)
        pltpu.sync_copy(ew_hbm.at[pl.ds(wid * CPT, CPT)], ew_sl)

        pltpu.sync_copy(zero_hbm.at[pl.ds(sid * RPT, RPT)],
                        acc.at[pl.ds(sid * RPT, RPT)])
        plsc.subcore_barrier()

        def gather(c, buf, sem):
            return pltpu.async_copy(p_hbm.at[src_sl.at[c]], buf, sem)

        def gather_wait(c, buf, sem):
            pltpu.make_async_copy(p_hbm.at[src_sl.at[c]], buf, sem).wait()

        def scale(c, buf):
            def grp(g, carry):
                ew16 = ew_sl[c, pl.ds(g * 16, 16)]
                for j in range(16):
                    wv = jnp.take_along_axis(
                        ew16, jnp.full((16,), j, jnp.int32), axis=0)
                    e = g * 16 + j
                    for f0 in range(0, F, 16):
                        buf[e, pl.ds(f0, 16)] = buf[e, pl.ds(f0, 16)] * wv
                return carry
            lax.fori_loop(0, C // 16, grp, 0)

        def scatter(c, buf, sem):
            return pltpu.async_copy(buf, acc.at[dst_sl.at[c]], sem, add=True)

        gather(0, buf_a, sga)

        def pair(q, carry):
            a = 2 * q
            b = a + 1
            gather(b, buf_b, sgb)
            gather_wait(a, buf_a, sga)
            scale(a, buf_a)
            da = scatter(a, buf_a, ssa)
            gather_wait(b, buf_b, sgb)
            scale(b, buf_b)
            db = scatter(b, buf_b, ssb)
            da.wait()
            nxt = jnp.minimum(a + 2, CPT - 1)
            gather(nxt, buf_a, sga)
            db.wait()
            return carry

        lax.fori_loop(0, CPT // 2, pair, 0)
        gather_wait(CPT - 1, buf_a, sga)
        plsc.subcore_barrier()

        pltpu.sync_copy(acc.at[pl.ds(sid * RPT, RPT)],
                        out_hbm.at[pl.ds(cid * NP + sid * RPT, RPT)])

    return segsum


_segsum16 = _make_segsum(16)
_segsum64 = _make_segsum(64)


def _tc1_body(x_ref, w_ref, b_ref, out_ref):
    out_ref[...] = jnp.maximum(x_ref[...] @ w_ref[...] + b_ref[...], 0.0)


def _tc2_body(parts_ref, h0_ref, w1r_ref, b1r_ref, w1s_ref,
              w2r_ref, b2r_ref, w2s_ref, p2_ref, r2_ref):
    agg = parts_ref[:N, :] + parts_ref[NP:NP + N, :]
    h1 = jnp.maximum(agg @ w1r_ref[...] + b1r_ref[...]
                     + h0_ref[...] @ w1s_ref[...], 0.0)
    p2_ref[...] = h1 @ w2r_ref[...]
    r2_ref[...] = h1 @ w2s_ref[...] + b2r_ref[...]


def _tc3_body(parts_ref, r_ref, w3r_ref, b3r_ref, w3s_ref, p3_ref, r3_ref):
    h2 = jnp.maximum(parts_ref[:N, :] + parts_ref[NP:NP + N, :] + r_ref[...], 0.0)
    p3_ref[...] = h2 @ w3r_ref[...]
    r3_ref[...] = h2 @ w3s_ref[...] + b3r_ref[...]


def _tc4_body(parts_ref, r_ref, batch_ref, wa_ref, ba_ref, wb_ref, bb_ref,
              out_ref):
    h3 = parts_ref[:N, :] + parts_ref[NP:NP + N, :] + r_ref[...]
    gids = lax.broadcasted_iota(jnp.int32, (N, G), 1)
    oh = (batch_ref[...] == gids).astype(jnp.float32)
    cnt = jnp.sum(oh, axis=0, keepdims=True)
    ohs = oh / jnp.maximum(cnt, 1.0)
    pooled = lax.dot_general(ohs, h3, (((0,), (0,)), ((), ())))
    h4 = jnp.maximum(pooled @ wa_ref[...] + ba_ref[...], 0.0)
    out_ref[...] = h4 @ wb_ref[...] + bb_ref[...]


def _tc_call(body, out_shapes, *args):
    return pl.pallas_call(
        body,
        out_shape=out_shapes,
    )(*args)


def kernel(x, edge_index, edge_weight, batch, W_lin1, b_lin1,
           W1_rel, b1_rel, W1_root,
           W2_rel, b2_rel, W2_root,
           W3_rel, b3_rel, W3_root,
           W_l2a, b_l2a, W_l2b, b_l2b):
    pad = EP - E
    src = jnp.pad(edge_index[0].astype(jnp.int32), (0, pad)).reshape(NROW, C)
    dst = jnp.pad(edge_index[1].astype(jnp.int32), (0, pad)).reshape(NROW, C)
    ew = jnp.pad(edge_weight.astype(jnp.float32), (0, pad)).reshape(NROW, C)
    batch2d = batch.astype(jnp.int32).reshape(N, 1)

    W1p = jnp.pad(W_lin1, ((0, 0), (0, 8)))
    b1p = jnp.pad(b_lin1, (0, 8)).reshape(1, 16)
    W1_rel_p = jnp.pad(W1_rel, ((0, 8), (0, 0)))
    W1_root_p = jnp.pad(W1_root, ((0, 8), (0, 0)))

    zero16 = jnp.zeros((NP, 16), jnp.float32)
    zero64 = jnp.zeros((NP, 64), jnp.float32)

    h0p = _tc_call(_tc1_body, jax.ShapeDtypeStruct((N, 16), jnp.float32),
                   x, W1p, b1p)

    parts1 = _segsum16(h0p, src, dst, ew, zero16)

    p2, r2 = _tc_call(
        _tc2_body,
        (jax.ShapeDtypeStruct((N, 64), jnp.float32),
         jax.ShapeDtypeStruct((N, 64), jnp.float32)),
        parts1, h0p, W1_rel_p, b1_rel.reshape(1, 64), W1_root_p,
        W2_rel, b2_rel.reshape(1, 64), W2_root)

    parts2 = _segsum64(p2, src, dst, ew, zero64)

    p3, r3 = _tc_call(
        _tc3_body,
        (jax.ShapeDtypeStruct((N, 64), jnp.float32),
         jax.ShapeDtypeStruct((N, 64), jnp.float32)),
        parts2, r2, W3_rel, b3_rel.reshape(1, 64), W3_root)

    parts3 = _segsum64(p3, src, dst, ew, zero64)

    out = _tc_call(
        _tc4_body,
        jax.ShapeDtypeStruct((G, b_l2b.shape[0]), jnp.float32),
        parts3, r3, batch2d, W_l2a, b_l2a.reshape(1, 32),
        W_l2b, b_l2b.reshape(1, b_l2b.shape[0]))
    return out


# speedup vs baseline: 8.4108x; 1.3868x over previous
import functools

import jax
import jax.numpy as jnp
from jax import lax
from jax.experimental import pallas as pl
from jax.experimental.pallas import tpu as pltpu
from jax.experimental.pallas import tpu_sc as plsc

N = 10000
E = 320000
G = 16
NC = 2
NS = 16
NW = NC * NS
C = 128
NROW = 2560
EP = NROW * C
CPT = NROW // NW
NP = 10240
RPT = NP // NS


def _make_segsum(F: int):
    mesh = plsc.VectorSubcoreMesh(
        core_axis_name="c", subcore_axis_name="s", num_cores=NC, num_subcores=NS
    )

    @functools.partial(
        pl.kernel,
        out_type=jax.ShapeDtypeStruct((NC * NP, F), jnp.float32),
        mesh=mesh,
        scratch_types=[
            pltpu.VMEM_SHARED((NP, F), jnp.float32),
            pltpu.VMEM((CPT, C), jnp.int32),
            pltpu.VMEM((CPT, C), jnp.int32),
            pltpu.VMEM((CPT, C), jnp.float32),
            pltpu.VMEM((C, F), jnp.float32),
            pltpu.VMEM((C, F), jnp.float32),
            pltpu.SemaphoreType.DMA,
            pltpu.SemaphoreType.DMA,
            pltpu.SemaphoreType.DMA,
            pltpu.SemaphoreType.DMA,
        ],
        compiler_params=pltpu.CompilerParams(use_tc_tiling_on_sc=False),
    )
    def segsum(p_hbm, src_hbm, dst_hbm, ew_hbm, zero_hbm, out_hbm,
               acc, src_sl, dst_sl, ew_sl, buf_a, buf_b,
               sga, sgb, ssa, ssb):
        cid = lax.axis_index("c")
        sid = lax.axis_index("s")
        wid = cid * NS + sid

        pltpu.sync_copy(src_hbm.at[pl.ds(wid * CPT, CPT)], src_sl)
        pltpu.sync_copy(dst_hbm.at[pl.ds(wid * CPT, CPT)], dst_sl)
        pltpu.sync_copy(ew_hbm.at[pl.ds(wid * CPT, CPT)], ew_sl)

        pltpu.sync_copy(zero_hbm.at[pl.ds(sid * RPT, RPT)],
                        acc.at[pl.ds(sid * RPT, RPT)])
        plsc.subcore_barrier()

        def gather(c, buf, sem):
            return pltpu.async_copy(p_hbm.at[src_sl.at[c]], buf, sem)

        def gather_wait(c, buf, sem):
            pltpu.make_async_copy(p_hbm.at[src_sl.at[c]], buf, sem).wait()

        def scale(c, buf):
            def grp(g, carry):
                ew16 = ew_sl[c, pl.ds(g * 16, 16)]
                for j in range(16):
                    wv = jnp.take_along_axis(
                        ew16, jnp.full((16,), j, jnp.int32), axis=0)
                    e = g * 16 + j
                    for f0 in range(0, F, 16):
                        buf[e, pl.ds(f0, 16)] = buf[e, pl.ds(f0, 16)] * wv
                return carry
            lax.fori_loop(0, C // 16, grp, 0)

        def scatter(c, buf, sem):
            return pltpu.async_copy(buf, acc.at[dst_sl.at[c]], sem, add=True)

        gather(0, buf_a, sga)

        def pair(q, carry):
            a = 2 * q
            b = a + 1
            gather(b, buf_b, sgb)
            gather_wait(a, buf_a, sga)
            scale(a, buf_a)
            da = scatter(a, buf_a, ssa)
            gather_wait(b, buf_b, sgb)
            scale(b, buf_b)
            db = scatter(b, buf_b, ssb)
            da.wait()
            nxt = jnp.minimum(a + 2, CPT - 1)
            gather(nxt, buf_a, sga)
            db.wait()
            return carry

        lax.fori_loop(0, CPT // 2, pair, 0)
        gather_wait(CPT - 1, buf_a, sga)
        plsc.subcore_barrier()

        pltpu.sync_copy(acc.at[pl.ds(sid * RPT, RPT)],
                        out_hbm.at[pl.ds(cid * NP + sid * RPT, RPT)])

    return segsum


_segsum16 = _make_segsum(16)
_segsum64 = _make_segsum(64)


def _tc1_body(x_ref, w_ref, b_ref, out_ref):
    out_ref[...] = jnp.maximum(x_ref[...] @ w_ref[...] + b_ref[...], 0.0)


def _tc2_body(parts_ref, h0_ref, w1r_ref, b1r_ref, w1s_ref,
              w2r_ref, b2r_ref, w2s_ref, p2_ref, r2_ref):
    agg = parts_ref[:N, :] + parts_ref[NP:NP + N, :]
    h1 = jnp.maximum(agg @ w1r_ref[...] + b1r_ref[...]
                     + h0_ref[...] @ w1s_ref[...], 0.0)
    p2_ref[...] = h1 @ w2r_ref[...]
    r2_ref[...] = h1 @ w2s_ref[...] + b2r_ref[...]


def _tc3_body(parts_ref, r_ref, w3r_ref, b3r_ref, w3s_ref, p3_ref, r3_ref):
    h2 = jnp.maximum(parts_ref[:N, :] + parts_ref[NP:NP + N, :] + r_ref[...], 0.0)
    p3_ref[...] = h2 @ w3r_ref[...]
    r3_ref[...] = h2 @ w3s_ref[...] + b3r_ref[...]


def _tc4_body(parts_ref, r_ref, batch_ref, wa_ref, ba_ref, wb_ref, bb_ref,
              out_ref):
    h3 = parts_ref[:N, :] + parts_ref[NP:NP + N, :] + r_ref[...]
    gids = lax.broadcasted_iota(jnp.int32, (N, G), 1)
    oh = (batch_ref[...] == gids).astype(jnp.float32)
    cnt = jnp.sum(oh, axis=0, keepdims=True)
    ohs = oh / jnp.maximum(cnt, 1.0)
    pooled = lax.dot_general(ohs, h3, (((0,), (0,)), ((), ())))
    h4 = jnp.maximum(pooled @ wa_ref[...] + ba_ref[...], 0.0)
    out_ref[...] = h4 @ wb_ref[...] + bb_ref[...]


def _tc_call(body, out_shapes, *args):
    return pl.pallas_call(
        body,
        out_shape=out_shapes,
    )(*args)


def kernel(x, edge_index, edge_weight, batch, W_lin1, b_lin1,
           W1_rel, b1_rel, W1_root,
           W2_rel, b2_rel, W2_root,
           W3_rel, b3_rel, W3_root,
           W_l2a, b_l2a, W_l2b, b_l2b):
    pad = EP - E
    spread = (jnp.arange(pad, dtype=jnp.int32) * 8) % N
    src = jnp.concatenate(
        [edge_index[0].astype(jnp.int32), spread]).reshape(NROW, C)
    dst = jnp.concatenate(
        [edge_index[1].astype(jnp.int32), spread]).reshape(NROW, C)
    ew = jnp.pad(edge_weight.astype(jnp.float32), (0, pad)).reshape(NROW, C)
    batch2d = batch.astype(jnp.int32).reshape(N, 1)

    W1p = jnp.pad(W_lin1, ((0, 0), (0, 8)))
    b1p = jnp.pad(b_lin1, (0, 8)).reshape(1, 16)
    W1_rel_p = jnp.pad(W1_rel, ((0, 8), (0, 0)))
    W1_root_p = jnp.pad(W1_root, ((0, 8), (0, 0)))

    zero16 = jnp.zeros((NP, 16), jnp.float32)
    zero64 = jnp.zeros((NP, 64), jnp.float32)

    h0p = _tc_call(_tc1_body, jax.ShapeDtypeStruct((N, 16), jnp.float32),
                   x, W1p, b1p)

    parts1 = _segsum16(h0p, src, dst, ew, zero16)

    p2, r2 = _tc_call(
        _tc2_body,
        (jax.ShapeDtypeStruct((N, 64), jnp.float32),
         jax.ShapeDtypeStruct((N, 64), jnp.float32)),
        parts1, h0p, W1_rel_p, b1_rel.reshape(1, 64), W1_root_p,
        W2_rel, b2_rel.reshape(1, 64), W2_root)

    parts2 = _segsum64(p2, src, dst, ew, zero64)

    p3, r3 = _tc_call(
        _tc3_body,
        (jax.ShapeDtypeStruct((N, 64), jnp.float32),
         jax.ShapeDtypeStruct((N, 64), jnp.float32)),
        parts2, r2, W3_rel, b3_rel.reshape(1, 64), W3_root)

    parts3 = _segsum64(p3, src, dst, ew, zero64)

    out = _tc_call(
        _tc4_body,
        jax.ShapeDtypeStruct((G, b_l2b.shape[0]), jnp.float32),
        parts3, r3, batch2d, W_l2a, b_l2a.reshape(1, 32),
        W_l2b, b_l2b.reshape(1, b_l2b.shape[0]))
    return out


# speedup vs baseline: 14.5587x; 1.7310x over previous
import functools

import jax
import jax.numpy as jnp
from jax import lax
from jax.experimental import pallas as pl
from jax.experimental.pallas import tpu as pltpu
from jax.experimental.pallas import tpu_sc as plsc

N = 10000
E = 320000
G = 16
NC = 2
NS = 16
NW = NC * NS
C = 128
NROW = 2560
EP = NROW * C
CPT = NROW // NW
NP = 10240
RPT = NP // NS


def _make_segsum(F: int):
    mesh = plsc.VectorSubcoreMesh(
        core_axis_name="c", subcore_axis_name="s", num_cores=NC, num_subcores=NS
    )

    @functools.partial(
        pl.kernel,
        out_type=jax.ShapeDtypeStruct((NC * NP, F), jnp.float32),
        mesh=mesh,
        scratch_types=[
            pltpu.VMEM_SHARED((NP, F), jnp.float32),
            pltpu.VMEM((CPT, C), jnp.int32),
            pltpu.VMEM((CPT, C), jnp.int32),
            pltpu.VMEM((CPT, C), jnp.float32),
            pltpu.VMEM((C, F), jnp.float32),
            pltpu.VMEM((C, F), jnp.float32),
            pltpu.SemaphoreType.DMA,
            pltpu.SemaphoreType.DMA,
            pltpu.SemaphoreType.DMA,
            pltpu.SemaphoreType.DMA,
        ],
        compiler_params=pltpu.CompilerParams(use_tc_tiling_on_sc=False),
    )
    def segsum(p_hbm, src_hbm, dst_hbm, ew_hbm, zero_hbm, out_hbm,
               acc, src_sl, dst_sl, ew_sl, buf_a, buf_b,
               sga, sgb, ssa, ssb):
        cid = lax.axis_index("c")
        sid = lax.axis_index("s")
        wid = cid * NS + sid

        pltpu.sync_copy(src_hbm.at[pl.ds(wid * CPT, CPT)], src_sl)
        pltpu.sync_copy(dst_hbm.at[pl.ds(wid * CPT, CPT)], dst_sl)
        pltpu.sync_copy(ew_hbm.at[pl.ds(wid * CPT, CPT)], ew_sl)

        pltpu.sync_copy(zero_hbm.at[pl.ds(sid * RPT, RPT)],
                        acc.at[pl.ds(sid * RPT, RPT)])
        plsc.subcore_barrier()

        def gather(c, buf, sem):
            return pltpu.async_copy(p_hbm.at[src_sl.at[c]], buf, sem)

        def gather_wait(c, buf, sem):
            pltpu.make_async_copy(p_hbm.at[src_sl.at[c]], buf, sem).wait()

        def scale(c, buf):
            for g in range(C // 16):
                ew16 = ew_sl[c, pl.ds(g * 16, 16)]
                for j in range(16):
                    wv = jnp.take_along_axis(
                        ew16, jnp.full((16,), j, jnp.int32), axis=0)
                    e = g * 16 + j
                    for f0 in range(0, F, 16):
                        buf[e, pl.ds(f0, 16)] = buf[e, pl.ds(f0, 16)] * wv

        def scatter(c, buf, sem):
            return pltpu.async_copy(buf, acc.at[dst_sl.at[c]], sem, add=True)

        gather(0, buf_a, sga)

        def pair(q, carry):
            a = 2 * q
            b = a + 1
            gather(b, buf_b, sgb)
            gather_wait(a, buf_a, sga)
            scale(a, buf_a)
            da = scatter(a, buf_a, ssa)
            gather_wait(b, buf_b, sgb)
            scale(b, buf_b)
            db = scatter(b, buf_b, ssb)
            da.wait()
            nxt = jnp.minimum(a + 2, CPT - 1)
            gather(nxt, buf_a, sga)
            db.wait()
            return carry

        lax.fori_loop(0, CPT // 2, pair, 0)
        gather_wait(CPT - 1, buf_a, sga)
        plsc.subcore_barrier()

        pltpu.sync_copy(acc.at[pl.ds(sid * RPT, RPT)],
                        out_hbm.at[pl.ds(cid * NP + sid * RPT, RPT)])

    return segsum


_segsum16 = _make_segsum(16)
_segsum64 = _make_segsum(64)


def _tc1_body(x_ref, w_ref, b_ref, out_ref):
    out_ref[...] = jnp.maximum(x_ref[...] @ w_ref[...] + b_ref[...], 0.0)


def _tc2_body(parts_ref, h0_ref, w1r_ref, b1r_ref, w1s_ref,
              w2r_ref, b2r_ref, w2s_ref, p2_ref, r2_ref):
    agg = parts_ref[:N, :] + parts_ref[NP:NP + N, :]
    h1 = jnp.maximum(agg @ w1r_ref[...] + b1r_ref[...]
                     + h0_ref[...] @ w1s_ref[...], 0.0)
    p2_ref[...] = h1 @ w2r_ref[...]
    r2_ref[...] = h1 @ w2s_ref[...] + b2r_ref[...]


def _tc3_body(parts_ref, r_ref, w3r_ref, b3r_ref, w3s_ref, p3_ref, r3_ref):
    h2 = jnp.maximum(parts_ref[:N, :] + parts_ref[NP:NP + N, :] + r_ref[...], 0.0)
    p3_ref[...] = h2 @ w3r_ref[...]
    r3_ref[...] = h2 @ w3s_ref[...] + b3r_ref[...]


def _tc4_body(parts_ref, r_ref, batch_ref, wa_ref, ba_ref, wb_ref, bb_ref,
              out_ref):
    h3 = parts_ref[:N, :] + parts_ref[NP:NP + N, :] + r_ref[...]
    gids = lax.broadcasted_iota(jnp.int32, (N, G), 1)
    oh = (batch_ref[...] == gids).astype(jnp.float32)
    cnt = jnp.sum(oh, axis=0, keepdims=True)
    ohs = oh / jnp.maximum(cnt, 1.0)
    pooled = lax.dot_general(ohs, h3, (((0,), (0,)), ((), ())))
    h4 = jnp.maximum(pooled @ wa_ref[...] + ba_ref[...], 0.0)
    out_ref[...] = h4 @ wb_ref[...] + bb_ref[...]


def _tc_call(body, out_shapes, *args):
    return pl.pallas_call(
        body,
        out_shape=out_shapes,
    )(*args)


def kernel(x, edge_index, edge_weight, batch, W_lin1, b_lin1,
           W1_rel, b1_rel, W1_root,
           W2_rel, b2_rel, W2_root,
           W3_rel, b3_rel, W3_root,
           W_l2a, b_l2a, W_l2b, b_l2b):
    pad = EP - E
    spread = (jnp.arange(pad, dtype=jnp.int32) * 8) % N
    src = jnp.concatenate(
        [edge_index[0].astype(jnp.int32), spread]).reshape(NROW, C)
    dst = jnp.concatenate(
        [edge_index[1].astype(jnp.int32), spread]).reshape(NROW, C)
    ew = jnp.pad(edge_weight.astype(jnp.float32), (0, pad)).reshape(NROW, C)
    batch2d = batch.astype(jnp.int32).reshape(N, 1)

    W1p = jnp.pad(W_lin1, ((0, 0), (0, 8)))
    b1p = jnp.pad(b_lin1, (0, 8)).reshape(1, 16)
    W1_rel_p = jnp.pad(W1_rel, ((0, 8), (0, 0)))
    W1_root_p = jnp.pad(W1_root, ((0, 8), (0, 0)))

    zero16 = jnp.zeros((NP, 16), jnp.float32)
    zero64 = jnp.zeros((NP, 64), jnp.float32)

    h0p = _tc_call(_tc1_body, jax.ShapeDtypeStruct((N, 16), jnp.float32),
                   x, W1p, b1p)

    parts1 = _segsum16(h0p, src, dst, ew, zero16)

    p2, r2 = _tc_call(
        _tc2_body,
        (jax.ShapeDtypeStruct((N, 64), jnp.float32),
         jax.ShapeDtypeStruct((N, 64), jnp.float32)),
        parts1, h0p, W1_rel_p, b1_rel.reshape(1, 64), W1_root_p,
        W2_rel, b2_rel.reshape(1, 64), W2_root)

    parts2 = _segsum64(p2, src, dst, ew, zero64)

    p3, r3 = _tc_call(
        _tc3_body,
        (jax.ShapeDtypeStruct((N, 64), jnp.float32),
         jax.ShapeDtypeStruct((N, 64), jnp.float32)),
        parts2, r2, W3_rel, b3_rel.reshape(1, 64), W3_root)

    parts3 = _segsum64(p3, src, dst, ew, zero64)

    out = _tc_call(
        _tc4_body,
        jax.ShapeDtypeStruct((G, b_l2b.shape[0]), jnp.float32),
        parts3, r3, batch2d, W_l2a, b_l2a.reshape(1, 32),
        W_l2b, b_l2b.reshape(1, b_l2b.shape[0]))
    return out


# speedup vs baseline: 15.1451x; 1.0403x over previous
import functools

import jax
import jax.numpy as jnp
from jax import lax
from jax.experimental import pallas as pl
from jax.experimental.pallas import tpu as pltpu
from jax.experimental.pallas import tpu_sc as plsc

N = 10000
E = 320000
G = 16
NC = 2
NS = 16
NW = NC * NS
C = 128
NROW = 2560
EP = NROW * C
CPT = NROW // NW
NP = 10240
RPT = NP // NS


def _make_segsum(F: int):
    mesh = plsc.VectorSubcoreMesh(
        core_axis_name="c", subcore_axis_name="s", num_cores=NC, num_subcores=NS
    )

    @functools.partial(
        pl.kernel,
        out_type=jax.ShapeDtypeStruct((NC * NP, F), jnp.float32),
        mesh=mesh,
        scratch_types=[
            pltpu.VMEM_SHARED((NP, F), jnp.float32),
            pltpu.VMEM((CPT, C), jnp.int32),
            pltpu.VMEM((CPT, C), jnp.int32),
            pltpu.VMEM((CPT, C), jnp.float32),
            pltpu.VMEM((C, F), jnp.float32),
            pltpu.VMEM((C, F), jnp.float32),
            pltpu.SemaphoreType.DMA,
            pltpu.SemaphoreType.DMA,
            pltpu.SemaphoreType.DMA,
            pltpu.SemaphoreType.DMA,
        ],
        compiler_params=pltpu.CompilerParams(use_tc_tiling_on_sc=False),
    )
    def segsum(p_hbm, src_hbm, dst_hbm, ew_hbm, zero_hbm, out_hbm,
               acc, src_sl, dst_sl, ew_sl, buf_a, buf_b,
               sga, sgb, ssa, ssb):
        cid = lax.axis_index("c")
        sid = lax.axis_index("s")
        wid = cid * NS + sid

        pltpu.sync_copy(src_hbm.at[pl.ds(wid * CPT, CPT)], src_sl)
        pltpu.sync_copy(dst_hbm.at[pl.ds(wid * CPT, CPT)], dst_sl)
        pltpu.sync_copy(ew_hbm.at[pl.ds(wid * CPT, CPT)], ew_sl)

        pltpu.sync_copy(zero_hbm.at[pl.ds(sid * RPT, RPT)],
                        acc.at[pl.ds(sid * RPT, RPT)])
        plsc.subcore_barrier()

        def gather(c, buf, sem):
            return pltpu.async_copy(p_hbm.at[src_sl.at[c]], buf, sem)

        def gather_wait(c, buf, sem):
            pltpu.make_async_copy(p_hbm.at[src_sl.at[c]], buf, sem).wait()

        def scale(c, buf):
            for g in range(C // 16):
                ew16 = ew_sl[c, pl.ds(g * 16, 16)]
                for j in range(16):
                    wv = jnp.take_along_axis(
                        ew16, jnp.full((16,), j, jnp.int32), axis=0)
                    e = g * 16 + j
                    for f0 in range(0, F, 16):
                        buf[e, pl.ds(f0, 16)] = buf[e, pl.ds(f0, 16)] * wv

        def scatter(c, buf, sem):
            return pltpu.async_copy(buf, acc.at[dst_sl.at[c]], sem, add=True)

        gather(0, buf_a, sga)

        def pair(q, carry):
            a = 2 * q
            b = a + 1
            gather(b, buf_b, sgb)
            gather_wait(a, buf_a, sga)
            scale(a, buf_a)
            da = scatter(a, buf_a, ssa)
            gather_wait(b, buf_b, sgb)
            da.wait()
            nxt = jnp.minimum(a + 2, CPT - 1)
            gather(nxt, buf_a, sga)
            scale(b, buf_b)
            db = scatter(b, buf_b, ssb)
            db.wait()
            return carry

        lax.fori_loop(0, CPT // 2, pair, 0)
        gather_wait(CPT - 1, buf_a, sga)
        plsc.subcore_barrier()

        pltpu.sync_copy(acc.at[pl.ds(sid * RPT, RPT)],
                        out_hbm.at[pl.ds(cid * NP + sid * RPT, RPT)])

    return segsum


_segsum16 = _make_segsum(16)
_segsum64 = _make_segsum(64)


def _tc1_body(x_ref, w_ref, b_ref, out_ref):
    out_ref[...] = jnp.maximum(x_ref[...] @ w_ref[...] + b_ref[...], 0.0)


def _tc2_body(parts_ref, h0_ref, w1r_ref, b1r_ref, w1s_ref,
              w2r_ref, b2r_ref, w2s_ref, p2_ref, r2_ref):
    agg = parts_ref[:N, :] + parts_ref[NP:NP + N, :]
    h1 = jnp.maximum(agg @ w1r_ref[...] + b1r_ref[...]
                     + h0_ref[...] @ w1s_ref[...], 0.0)
    p2_ref[...] = h1 @ w2r_ref[...]
    r2_ref[...] = h1 @ w2s_ref[...] + b2r_ref[...]


def _tc3_body(parts_ref, r_ref, w3r_ref, b3r_ref, w3s_ref, p3_ref, r3_ref):
    h2 = jnp.maximum(parts_ref[:N, :] + parts_ref[NP:NP + N, :] + r_ref[...], 0.0)
    p3_ref[...] = h2 @ w3r_ref[...]
    r3_ref[...] = h2 @ w3s_ref[...] + b3r_ref[...]


def _tc4_body(parts_ref, r_ref, batch_ref, wa_ref, ba_ref, wb_ref, bb_ref,
              out_ref):
    h3 = parts_ref[:N, :] + parts_ref[NP:NP + N, :] + r_ref[...]
    gids = lax.broadcasted_iota(jnp.int32, (N, G), 1)
    oh = (batch_ref[...] == gids).astype(jnp.float32)
    cnt = jnp.sum(oh, axis=0, keepdims=True)
    ohs = oh / jnp.maximum(cnt, 1.0)
    pooled = lax.dot_general(ohs, h3, (((0,), (0,)), ((), ())))
    h4 = jnp.maximum(pooled @ wa_ref[...] + ba_ref[...], 0.0)
    out_ref[...] = h4 @ wb_ref[...] + bb_ref[...]


def _tc_call(body, out_shapes, *args):
    return pl.pallas_call(
        body,
        out_shape=out_shapes,
    )(*args)


def kernel(x, edge_index, edge_weight, batch, W_lin1, b_lin1,
           W1_rel, b1_rel, W1_root,
           W2_rel, b2_rel, W2_root,
           W3_rel, b3_rel, W3_root,
           W_l2a, b_l2a, W_l2b, b_l2b):
    pad = EP - E
    spread = (jnp.arange(pad, dtype=jnp.int32) * 8) % N
    src = jnp.concatenate(
        [edge_index[0].astype(jnp.int32), spread]).reshape(NROW, C)
    dst = jnp.concatenate(
        [edge_index[1].astype(jnp.int32), spread]).reshape(NROW, C)
    ew = jnp.pad(edge_weight.astype(jnp.float32), (0, pad)).reshape(NROW, C)
    batch2d = batch.astype(jnp.int32).reshape(N, 1)

    W1p = jnp.pad(W_lin1, ((0, 0), (0, 8)))
    b1p = jnp.pad(b_lin1, (0, 8)).reshape(1, 16)
    W1_rel_p = jnp.pad(W1_rel, ((0, 8), (0, 0)))
    W1_root_p = jnp.pad(W1_root, ((0, 8), (0, 0)))

    zero16 = jnp.zeros((NP, 16), jnp.float32)
    zero64 = jnp.zeros((NP, 64), jnp.float32)

    h0p = _tc_call(_tc1_body, jax.ShapeDtypeStruct((N, 16), jnp.float32),
                   x, W1p, b1p)

    parts1 = _segsum16(h0p, src, dst, ew, zero16)

    p2, r2 = _tc_call(
        _tc2_body,
        (jax.ShapeDtypeStruct((N, 64), jnp.float32),
         jax.ShapeDtypeStruct((N, 64), jnp.float32)),
        parts1, h0p, W1_rel_p, b1_rel.reshape(1, 64), W1_root_p,
        W2_rel, b2_rel.reshape(1, 64), W2_root)

    parts2 = _segsum64(p2, src, dst, ew, zero64)

    p3, r3 = _tc_call(
        _tc3_body,
        (jax.ShapeDtypeStruct((N, 64), jnp.float32),
         jax.ShapeDtypeStruct((N, 64), jnp.float32)),
        parts2, r2, W3_rel, b3_rel.reshape(1, 64), W3_root)

    parts3 = _segsum64(p3, src, dst, ew, zero64)

    out = _tc_call(
        _tc4_body,
        jax.ShapeDtypeStruct((G, b_l2b.shape[0]), jnp.float32),
        parts3, r3, batch2d, W_l2a, b_l2a.reshape(1, 32),
        W_l2b, b_l2b.reshape(1, b_l2b.shape[0]))
    return out
